# baseline (speedup 1.0000x reference)
import functools

import jax
import jax.numpy as jnp
from jax.experimental import pallas as pl
from jax.experimental.pallas import tpu as pltpu

_B = 64
_BB = 32


_CB = 2


def _fused_body(t_ref, sac_ref, somac_ref, x_ref, n_ref, o_ref):
    i = pl.program_id(0)
    for r in range(_BB):
        ti = t_ref[i * _BB + r]
        c1 = sac_ref[ti]
        c2 = somac_ref[ti]
        o_ref[r] = c1 * x_ref[r] + c2 * n_ref[r]


@jax.jit
def _tc_fused(t, sac, somac, x, n):
    blk = (_BB, _CB) + x.shape[2:]
    imap = lambda i, j, *_: (i, j, 0, 0)
    grid_spec = pltpu.PrefetchScalarGridSpec(
        num_scalar_prefetch=3,
        grid=(_B // _BB, x.shape[1] // _CB),
        in_specs=[
            pl.BlockSpec(blk, imap),
            pl.BlockSpec(blk, imap),
        ],
        out_specs=pl.BlockSpec(blk, imap),
    )
    return pl.pallas_call(
        _fused_body,
        grid_spec=grid_spec,
        out_shape=jax.ShapeDtypeStruct(x.shape, jnp.float32),
        compiler_params=pltpu.CompilerParams(
            dimension_semantics=("parallel", "parallel")),
    )(t, sac, somac, x, n)


def kernel(x_start, noise, sqrt_alphas_cumprod, sqrt_one_minus_alphas_cumprod, t):
    return _tc_fused(t, sqrt_alphas_cumprod, sqrt_one_minus_alphas_cumprod,
                     x_start, noise)


# speedup vs baseline: 1.0360x; 1.0360x over previous
import jax
import jax.numpy as jnp
from jax.experimental import pallas as pl
from jax.experimental.pallas import tpu as pltpu

_B = 64
_R = 8
_C = _B // _R
_NBUF = 4


def _ring_body(t_s, sac_s, somac_s, x_hbm, n_hbm, o_hbm,
               xb, nb, ob, insem, outsem):
    def in_cp(c):
        slot = c % _NBUF
        return (
            pltpu.make_async_copy(x_hbm.at[pl.ds(c * _R, _R)], xb.at[slot],
                                  insem.at[slot, 0]),
            pltpu.make_async_copy(n_hbm.at[pl.ds(c * _R, _R)], nb.at[slot],
                                  insem.at[slot, 1]),
        )

    def out_cp(c):
        slot = c % _NBUF
        return pltpu.make_async_copy(ob.at[slot], o_hbm.at[pl.ds(c * _R, _R)],
                                     outsem.at[slot])

    for c in range(_NBUF):
        cx, cn = in_cp(c)
        cx.start()
        cn.start()
    for c in range(_C):
        slot = c % _NBUF
        cx, cn = in_cp(c)
        cx.wait()
        cn.wait()
        if c >= _NBUF:
            out_cp(c - _NBUF).wait()
        for r in range(_R):
            ti = t_s[c * _R + r]
            ob[slot, r] = sac_s[ti] * xb[slot, r] + somac_s[ti] * nb[slot, r]
        out_cp(c).start()
        if c + _NBUF < _C:
            nx, nn = in_cp(c + _NBUF)
            nx.start()
            nn.start()
    for c in range(_C - _NBUF, _C):
        out_cp(c).wait()


@jax.jit
def _tc_ring(t, sac, somac, x, n):
    smem = pl.BlockSpec(memory_space=pltpu.SMEM)
    hbm = pl.BlockSpec(memory_space=pl.ANY)
    row = x.shape[1:]
    return pl.pallas_call(
        _ring_body,
        in_specs=[smem, smem, smem, hbm, hbm],
        out_specs=hbm,
        out_shape=jax.ShapeDtypeStruct(x.shape, jnp.float32),
        scratch_shapes=[
            pltpu.VMEM((_NBUF, _R) + row, jnp.float32),
            pltpu.VMEM((_NBUF, _R) + row, jnp.float32),
            pltpu.VMEM((_NBUF, _R) + row, jnp.float32),
            pltpu.SemaphoreType.DMA((_NBUF, 2)),
            pltpu.SemaphoreType.DMA((_NBUF,)),
        ],
    )(t, sac, somac, x, n)


def kernel(x_start, noise, sqrt_alphas_cumprod, sqrt_one_minus_alphas_cumprod, t):
    return _tc_ring(t, sqrt_alphas_cumprod, sqrt_one_minus_alphas_cumprod,
                    x_start, noise)
